# Initial kernel scaffold; baseline (speedup 1.0000x reference)
import jax
import jax.numpy as jnp
from jax.experimental import pallas as pl


def kernel(nfeat, efeat, edge_index, node_graph_id, W0, b0, We1, be1, We2, be2, conv_b, gru_Wih, gru_Whh, gru_bih, gru_bhh, lstm_Wih, lstm_Whh, lstm_bih, lstm_bhh, fc1_W, fc1_b, fc2_W, fc2_b):
    raise NotImplementedError("write your pallas kernel here")


# speedup vs baseline: 1.4133x; 1.4133x over previous
import functools

import jax
import jax.numpy as jnp
from jax import lax
from jax.experimental import pallas as pl
from jax.experimental.pallas import tpu as pltpu
from jax.experimental.pallas import tpu_sc as plsc

N = 10000
E = 160000
B = 64
IN = 128
H = 16
EF = 5
HH = H * H

NC = 2
NS = 16
NW = NC * NS

CHUNK = 128
CHUNKS_PER_W = 40
EDGES_PER_W = CHUNKS_PER_W * CHUNK
E_PAD = NW * EDGES_PER_W
ACC_ROWS = 10240
ROWS_PER_TILE = ACC_ROWS // NS
DST_PAD_ROW = N + 8

_HIGH = lax.Precision.HIGHEST

_sc_mesh = plsc.VectorSubcoreMesh(core_axis_name="c", subcore_axis_name="s")
_sc_params = pltpu.CompilerParams(use_tc_tiling_on_sc=False)


@functools.partial(
    pl.kernel,
    out_type=jax.ShapeDtypeStruct((E_PAD, H), jnp.float32),
    mesh=_sc_mesh,
    compiler_params=_sc_params,
    scratch_types=[
        pltpu.VMEM((CHUNKS_PER_W, CHUNK), jnp.int32),
        pltpu.VMEM((EDGES_PER_W, H), jnp.float32),
        pltpu.SemaphoreType.DMA,
    ],
)
def _sc_gather(table_hbm, src_hbm, out_hbm, idx_v, rows_v, sem):
    wid = lax.axis_index("s") * NC + lax.axis_index("c")
    pltpu.sync_copy(src_hbm.at[pl.ds(wid * CHUNKS_PER_W, CHUNKS_PER_W)], idx_v)

    def body(j, carry):
        pltpu.async_copy(
            table_hbm.at[idx_v.at[j]],
            rows_v.at[pl.ds(j * CHUNK, CHUNK)],
            sem,
        ).wait()
        return carry

    lax.fori_loop(0, CHUNKS_PER_W, body, 0)
    pltpu.sync_copy(rows_v, out_hbm.at[pl.ds(wid * EDGES_PER_W, EDGES_PER_W)])


@functools.partial(
    pl.kernel,
    out_type=jax.ShapeDtypeStruct((NC, ACC_ROWS, H), jnp.float32),
    mesh=_sc_mesh,
    compiler_params=_sc_params,
    scratch_types=[
        pltpu.VMEM((CHUNKS_PER_W, CHUNK), jnp.int32),
        pltpu.VMEM((EDGES_PER_W, H), jnp.float32),
        pltpu.VMEM((ROWS_PER_TILE, H), jnp.float32),
        pltpu.VMEM_SHARED((ACC_ROWS, H), jnp.float32),
        pltpu.SemaphoreType.DMA,
    ],
)
def _sc_scatter(msg_hbm, dst_hbm, out_hbm, idx_v, msg_v, zero_v, acc_sh, sem):
    cid = lax.axis_index("c")
    sid = lax.axis_index("s")
    wid = sid * NC + cid

    def zbody(r, carry):
        zero_v[r, :] = jnp.zeros((H,), jnp.float32)
        return carry

    lax.fori_loop(0, ROWS_PER_TILE, zbody, 0)
    pltpu.sync_copy(zero_v, acc_sh.at[pl.ds(sid * ROWS_PER_TILE, ROWS_PER_TILE)])
    plsc.subcore_barrier()

    pltpu.sync_copy(dst_hbm.at[pl.ds(wid * CHUNKS_PER_W, CHUNKS_PER_W)], idx_v)
    pltpu.sync_copy(msg_hbm.at[pl.ds(wid * EDGES_PER_W, EDGES_PER_W)], msg_v)

    def body(j, carry):
        pltpu.sync_copy(
            msg_v.at[pl.ds(j * CHUNK, CHUNK)],
            acc_sh.at[idx_v.at[j]],
            add=True,
        )
        return carry

    lax.fori_loop(0, CHUNKS_PER_W, body, 0)
    plsc.subcore_barrier()
    pltpu.sync_copy(
        acc_sh.at[pl.ds(sid * ROWS_PER_TILE, ROWS_PER_TILE)],
        out_hbm.at[cid, pl.ds(sid * ROWS_PER_TILE, ROWS_PER_TILE)],
    )


def _lin0_body(x_ref, w_ref, b_ref, o_ref):
    o_ref[...] = jax.nn.relu(
        jnp.dot(x_ref[...], w_ref[...], precision=_HIGH,
                preferred_element_type=jnp.float32) + b_ref[...])


def _lin0(nfeat, W0, b0):
    bn = 2000
    return pl.pallas_call(
        _lin0_body,
        grid=(N // bn,),
        in_specs=[
            pl.BlockSpec((bn, IN), lambda i: (i, 0)),
            pl.BlockSpec((IN, H), lambda i: (0, 0)),
            pl.BlockSpec((1, H), lambda i: (0, 0)),
        ],
        out_specs=pl.BlockSpec((bn, H), lambda i: (i, 0)),
        out_shape=jax.ShapeDtypeStruct((N, H), jnp.float32),
    )(nfeat, W0, b0.reshape(1, H))


def _msg_body(ef_ref, x_ref, we1_ref, be1_ref, we2_ref, be2_ref, o_ref):
    z = jax.nn.relu(
        jnp.dot(ef_ref[...], we1_ref[...], precision=_HIGH,
                preferred_element_type=jnp.float32) + be1_ref[...])
    a = jnp.dot(z, we2_ref[...], precision=_HIGH,
                preferred_element_type=jnp.float32) + be2_ref[...]
    col = lax.broadcasted_iota(jnp.int32, (H, HH), 1)
    row = lax.broadcasted_iota(jnp.int32, (H, HH), 0)
    t_mat = (col // H == row).astype(jnp.float32)
    xrep = jnp.dot(x_ref[...], t_mat, precision=_HIGH,
                   preferred_element_type=jnp.float32)
    row2 = lax.broadcasted_iota(jnp.int32, (HH, H), 0)
    col2 = lax.broadcasted_iota(jnp.int32, (HH, H), 1)
    s_mat = (row2 % H == col2).astype(jnp.float32)
    o_ref[...] = jnp.dot(xrep * a, s_mat, precision=_HIGH,
                         preferred_element_type=jnp.float32)


def _msg(ef_p, xsrc, We1, be1, We2, be2):
    be = 2048
    return pl.pallas_call(
        _msg_body,
        grid=(E_PAD // be,),
        in_specs=[
            pl.BlockSpec((be, EF), lambda i: (i, 0)),
            pl.BlockSpec((be, H), lambda i: (i, 0)),
            pl.BlockSpec((EF, IN), lambda i: (0, 0)),
            pl.BlockSpec((1, IN), lambda i: (0, 0)),
            pl.BlockSpec((IN, HH), lambda i: (0, 0)),
            pl.BlockSpec((1, HH), lambda i: (0, 0)),
        ],
        out_specs=pl.BlockSpec((be, H), lambda i: (i, 0)),
        out_shape=jax.ShapeDtypeStruct((E_PAD, H), jnp.float32),
    )(ef_p, xsrc, We1, be1.reshape(1, IN), We2, be2.reshape(1, HH))


def _gru_body(aggp_ref, degp_ref, h_ref, convb_ref, wih_ref, whh_ref,
              bih_ref, bhh_ref, o_ref):
    agg = aggp_ref[0] + aggp_ref[1]
    deg = jnp.maximum(degp_ref[0] + degp_ref[1], 1.0)
    m = jax.nn.relu(agg / deg + convb_ref[...])
    h = h_ref[...]
    gi = jnp.dot(m, wih_ref[...], precision=_HIGH,
                 preferred_element_type=jnp.float32) + bih_ref[...]
    gh = jnp.dot(h, whh_ref[...], precision=_HIGH,
                 preferred_element_type=jnp.float32) + bhh_ref[...]
    r = jax.nn.sigmoid(gi[:, 0:H] + gh[:, 0:H])
    z = jax.nn.sigmoid(gi[:, H:2 * H] + gh[:, H:2 * H])
    n = jnp.tanh(gi[:, 2 * H:3 * H] + r * gh[:, 2 * H:3 * H])
    o_ref[...] = (1.0 - z) * n + z * h


def _gru(aggp, degp, h, conv_b, wih_t, whh_t, bih, bhh):
    bn = 2000
    return pl.pallas_call(
        _gru_body,
        grid=(N // bn,),
        in_specs=[
            pl.BlockSpec((NC, bn, H), lambda i: (0, i, 0)),
            pl.BlockSpec((NC, bn, H), lambda i: (0, i, 0)),
            pl.BlockSpec((bn, H), lambda i: (i, 0)),
            pl.BlockSpec((1, H), lambda i: (0, 0)),
            pl.BlockSpec((H, 3 * H), lambda i: (0, 0)),
            pl.BlockSpec((H, 3 * H), lambda i: (0, 0)),
            pl.BlockSpec((1, 3 * H), lambda i: (0, 0)),
            pl.BlockSpec((1, 3 * H), lambda i: (0, 0)),
        ],
        out_specs=pl.BlockSpec((bn, H), lambda i: (i, 0)),
        out_shape=jax.ShapeDtypeStruct((N, H), jnp.float32),
    )(aggp, degp, h, conv_b.reshape(1, H), wih_t, whh_t,
      bih.reshape(1, 3 * H), bhh.reshape(1, 3 * H))


def _s2s_body(h_ref, gid_ref, wih_ref, whh_ref, bih_ref, bhh_ref,
              fc1w_ref, fc1b_ref, fc2w_ref, fc2b_ref, o_ref):
    w = h_ref[...]
    gid = gid_ref[...]
    gt = (gid == lax.broadcasted_iota(jnp.int32, (N, B), 1)).astype(jnp.float32)
    q_star = jnp.zeros((B, 2 * H), jnp.float32)
    lh = jnp.zeros((B, H), jnp.float32)
    lc = jnp.zeros((B, H), jnp.float32)
    for _ in range(3):
        g = (jnp.dot(q_star, wih_ref[...], precision=_HIGH,
                     preferred_element_type=jnp.float32) + bih_ref[...]
             + jnp.dot(lh, whh_ref[...], precision=_HIGH,
                       preferred_element_type=jnp.float32) + bhh_ref[...])
        i_g = jax.nn.sigmoid(g[:, 0:H])
        f_g = jax.nn.sigmoid(g[:, H:2 * H])
        c_g = jnp.tanh(g[:, 2 * H:3 * H])
        o_g = jax.nn.sigmoid(g[:, 3 * H:4 * H])
        lc = f_g * lc + i_g * c_g
        lh = o_g * jnp.tanh(lc)
        q = lh
        e1 = jnp.dot(w, q.T, precision=_HIGH,
                     preferred_element_type=jnp.float32)
        e = jnp.sum(e1 * gt, axis=1, keepdims=True)
        emax = jnp.max(jnp.where(gt > 0.0, jnp.broadcast_to(e, (N, B)),
                                 -jnp.inf), axis=0, keepdims=True)
        emax = jnp.maximum(emax, -1e30)
        ex = jnp.exp(e - jnp.sum(gt * emax, axis=1, keepdims=True))
        denom = jnp.sum(gt * ex, axis=0, keepdims=True)
        alpha = ex / jnp.maximum(jnp.sum(gt * denom, axis=1, keepdims=True),
                                 1e-30)
        readout = lax.dot_general(gt, alpha * w, (((0,), (0,)), ((), ())),
                                  precision=_HIGH,
                                  preferred_element_type=jnp.float32)
        q_star = jnp.concatenate([q, readout], axis=1)
    hid1 = jax.nn.relu(
        jnp.dot(q_star, fc1w_ref[...], precision=_HIGH,
                preferred_element_type=jnp.float32) + fc1b_ref[...])
    o_ref[...] = (jnp.dot(hid1, fc2w_ref[...], precision=_HIGH,
                          preferred_element_type=jnp.float32) + fc2b_ref[...])


def _s2s(h, gid2d, lstm_Wih, lstm_Whh, lstm_bih, lstm_bhh,
         fc1_W, fc1_b, fc2_W, fc2_b):
    return pl.pallas_call(
        _s2s_body,
        grid=(1,),
        in_specs=[
            pl.BlockSpec((N, H), lambda i: (0, 0)),
            pl.BlockSpec((N, 1), lambda i: (0, 0)),
            pl.BlockSpec((2 * H, 4 * H), lambda i: (0, 0)),
            pl.BlockSpec((H, 4 * H), lambda i: (0, 0)),
            pl.BlockSpec((1, 4 * H), lambda i: (0, 0)),
            pl.BlockSpec((1, 4 * H), lambda i: (0, 0)),
            pl.BlockSpec((2 * H, H), lambda i: (0, 0)),
            pl.BlockSpec((1, H), lambda i: (0, 0)),
            pl.BlockSpec((H, 1), lambda i: (0, 0)),
            pl.BlockSpec((1, 1), lambda i: (0, 0)),
        ],
        out_specs=pl.BlockSpec((B, 1), lambda i: (0, 0)),
        out_shape=jax.ShapeDtypeStruct((B, 1), jnp.float32),
    )(h, gid2d, lstm_Wih.T, lstm_Whh.T, lstm_bih.reshape(1, 4 * H),
      lstm_bhh.reshape(1, 4 * H), fc1_W, fc1_b.reshape(1, H), fc2_W,
      fc2_b.reshape(1, 1))


def kernel(nfeat, efeat, edge_index, node_graph_id,
           W0, b0, We1, be1, We2, be2, conv_b,
           gru_Wih, gru_Whh, gru_bih, gru_bhh,
           lstm_Wih, lstm_Whh, lstm_bih, lstm_bhh,
           fc1_W, fc1_b, fc2_W, fc2_b):
    src = edge_index[0]
    dst = edge_index[1]
    pad = E_PAD - E
    src_p = jnp.concatenate(
        [src, jnp.zeros((pad,), jnp.int32)]).reshape(E_PAD // CHUNK, CHUNK)
    dst_p = jnp.concatenate(
        [dst, jnp.full((pad,), DST_PAD_ROW, jnp.int32)]
    ).reshape(E_PAD // CHUNK, CHUNK)
    ef_p = jnp.concatenate(
        [efeat, jnp.zeros((pad, EF), jnp.float32)], axis=0)

    h = _lin0(nfeat, W0, b0)
    degp = _sc_scatter(jnp.ones((E_PAD, H), jnp.float32), dst_p)[:, :N, :]
    wih_t = gru_Wih.T
    whh_t = gru_Whh.T
    for _ in range(3):
        xsrc = _sc_gather(h, src_p)
        msg = _msg(ef_p, xsrc, We1, be1, We2, be2)
        aggp = _sc_scatter(msg, dst_p)[:, :N, :]
        h = _gru(aggp, degp, h, conv_b, wih_t, whh_t, gru_bih, gru_bhh)

    pred = _s2s(h, node_graph_id.reshape(N, 1).astype(jnp.int32),
                lstm_Wih, lstm_Whh, lstm_bih, lstm_bhh,
                fc1_W, fc1_b, fc2_W, fc2_b)
    return pred.reshape(-1)
